# baseline (speedup 1.0000x reference)
import functools

import jax
import jax.numpy as jnp
from jax import lax
from jax.experimental import pallas as pl
from jax.experimental.pallas import tpu as pltpu
from jax.experimental.pallas import tpu_sc as plsc

B = 1024
L = 50
V = 100000
R = 16
D = 128
F = 256
E = 8
K = 2
H1 = 512
H2 = 512
O = 256
C = 16

N_ROWS = B * L
_NC = 2
_NS = 16
_NW = _NC * _NS
ROWS_PER_W = N_ROWS // _NW
CHUNK = 80
NCHUNK = ROWS_PER_W // CHUNK

_HI = lax.Precision.HIGHEST


def _b16(a):
    return a.astype(jnp.bfloat16).astype(jnp.float32)


NBUF = 4


def _sc_gather_body(table_hbm, idx_hbm, out_hbm, *scr):
    wid = lax.axis_index("s") * _NC + lax.axis_index("c")
    base = wid * ROWS_PER_W
    idxs = scr[0:NBUF]
    rows = scr[NBUF:2 * NBUF]
    gsems = scr[2 * NBUF:3 * NBUF]
    wsems = scr[3 * NBUF:4 * NBUF]

    def fire(j, s):
        off = base + j * CHUNK
        pltpu.sync_copy(idx_hbm.at[pl.ds(off, CHUNK)], idxs[s])
        return pltpu.async_copy(table_hbm.at[idxs[s]], rows[s], gsems[s])

    cps = [None] * NBUF
    wcs = [None] * NBUF
    for j in range(min(NBUF - 1, NCHUNK)):
        cps[j] = fire(j, j)
    for j in range(NCHUNK):
        s = j % NBUF
        cps[s].wait()
        wcs[s] = pltpu.async_copy(
            rows[s], out_hbm.at[pl.ds(base + j * CHUNK, CHUNK)], wsems[s])
        nj = j + NBUF - 1
        if nj < NCHUNK:
            sn = nj % NBUF
            if wcs[sn] is not None:
                wcs[sn].wait()
            cps[sn] = fire(nj, sn)
    for s in range(NBUF):
        if wcs[s] is not None:
            wcs[s].wait()


@functools.cache
def _build_sc_gather():
    return functools.partial(
        pl.kernel,
        mesh=plsc.VectorSubcoreMesh(core_axis_name="c",
                                    subcore_axis_name="s"),
        out_type=jax.ShapeDtypeStruct((N_ROWS, D), jnp.float32),
        scratch_types=(
            [pltpu.VMEM((CHUNK,), jnp.int32) for _ in range(NBUF)]
            + [pltpu.VMEM((CHUNK, D), jnp.float32) for _ in range(NBUF)]
            + [pltpu.SemaphoreType.DMA for _ in range(2 * NBUF)]
        ),
    )(_sc_gather_body)


NRB = 3200


def _proj_body(xg_ref, ridx_ref, rt_ref, WqT_ref, bq_ref, Wk_ref, bk_ref,
               Wv_ref, bv_ref, gctx_ref, score_ref, values_ref):
    f32 = jnp.float32
    oh = (ridx_ref[:] == lax.broadcasted_iota(jnp.int32, (1, R), 1))
    role = jnp.dot(oh.astype(f32), rt_ref[:], preferred_element_type=f32,
                   precision=_HI)
    x2 = xg_ref[:] + role
    q_col = jnp.dot(WqT_ref[:], gctx_ref[:],
                    preferred_element_type=f32) + bq_ref[:]
    keys = jnp.dot(x2, Wk_ref[:], preferred_element_type=f32) + bk_ref[:]
    score_ref[:] = jnp.dot(keys, q_col, preferred_element_type=f32)
    values_ref[:] = (jnp.dot(x2, Wv_ref[:], preferred_element_type=f32)
                     + bv_ref[:]).astype(jnp.bfloat16)


BB = 128


def _route_body(score_ref, val_ref, lng_ref, lnb_ref, Wg_ref, bg_ref,
                eb_ref, w_ref, topk_ref):
    f32 = jnp.float32
    score = score_ref[:]
    m = jnp.max(score, axis=1, keepdims=True)
    ex = jnp.exp(score - m)
    attn = ex / jnp.sum(ex, axis=1, keepdims=True)
    attn16 = _b16(attn)
    parts = [jnp.zeros((BB, D), f32) for _ in range(4)]
    for l in range(L):
        parts[l % 4] = (parts[l % 4]
                        + val_ref[:, l, :].astype(f32) * attn16[:, l:l + 1])
    pooled = (parts[0] + parts[1]) + (parts[2] + parts[3])
    mu = jnp.mean(pooled, axis=1, keepdims=True)
    cent = pooled - mu
    var = jnp.mean(cent * cent, axis=1, keepdims=True)
    ri = cent / jnp.sqrt(var + 1e-5) * lng_ref[:] + lnb_ref[:]

    gate = jnp.dot(ri, Wg_ref[:], preferred_element_type=f32) + bg_ref[:]
    sel = gate + eb_ref[:]
    iota_e = lax.broadcasted_iota(jnp.int32, (BB, E), 1)
    m1 = jnp.max(sel, axis=1, keepdims=True)
    i1 = jnp.min(jnp.where(sel == m1, iota_e, E), axis=1, keepdims=True)
    sel2 = jnp.where(iota_e == i1, -jnp.inf, sel)
    m2 = jnp.max(sel2, axis=1, keepdims=True)
    i2 = jnp.min(jnp.where(sel2 == m2, iota_e, E), axis=1, keepdims=True)
    g1 = jnp.sum(jnp.where(iota_e == i1, gate, 0.0), axis=1, keepdims=True)
    g2 = jnp.sum(jnp.where(iota_e == i2, gate, 0.0), axis=1, keepdims=True)
    mx = jnp.maximum(g1, g2)
    e1 = jnp.exp(g1 - mx)
    e2 = jnp.exp(g2 - mx)
    p1 = e1 / (e1 + e2)
    p2 = e2 / (e1 + e2)
    w_ref[:] = (p1 * (iota_e == i1).astype(f32)
                + p2 * (iota_e == i2).astype(f32))
    topk_ref[:] = jnp.concatenate([i1, i2], axis=1)


def _expert_body(feat_ref, w_ref, W1_ref, b1_ref, W2_ref, b2_ref,
                 W3_ref, b3_ref, Wp_ref, bp_ref, logits_ref, acc_ref):
    f32 = jnp.float32
    e = pl.program_id(0)
    h1 = jnp.maximum(
        jnp.dot(feat_ref[:], W1_ref[0], preferred_element_type=f32)
        + b1_ref[0], 0.0)
    h2 = jnp.maximum(
        jnp.dot(h1, W2_ref[0], preferred_element_type=f32) + b2_ref[0], 0.0)
    h3 = jnp.dot(h2, W3_ref[0], preferred_element_type=f32) + b3_ref[0]
    iota_e = lax.broadcasted_iota(jnp.int32, (B, E), 1)
    wcol = jnp.sum(jnp.where(iota_e == e, w_ref[:], 0.0), axis=1,
                   keepdims=True)

    @pl.when(e == 0)
    def _():
        acc_ref[:] = jnp.zeros((B, O), f32)

    acc_ref[:] += wcol * h3

    @pl.when(e == E - 1)
    def _():
        logits_ref[:] = (jnp.dot(acc_ref[:], Wp_ref[:],
                                 preferred_element_type=f32) + bp_ref[:])


def _row(v):
    return v.reshape(1, -1)


def kernel(ipc_indices, role_indices, bibliometric_features, ipc_table,
           role_table, Wq, bq, Wk, bk, Wv, bv, gctx, ln_g, ln_b, Wg, bg,
           expert_biases, W1, b1, W2, b2, W3, b3, Wp, bp):
    idx_flat = ipc_indices.reshape(-1).astype(jnp.int32)
    xg = _build_sc_gather()(ipc_table, idx_flat)

    full = lambda shape: pl.BlockSpec(shape, lambda i: (0,) * len(shape))
    score_col, values = pl.pallas_call(
        _proj_body,
        grid=(N_ROWS // NRB,),
        in_specs=[
            pl.BlockSpec((NRB, D), lambda i: (i, 0)),
            pl.BlockSpec((NRB, 1), lambda i: (i, 0)),
            full((R, D)), full((D, D)), full((D, 1)), full((D, D)),
            full((1, D)), full((D, D)), full((1, D)), full((D, 1)),
        ],
        out_specs=[
            pl.BlockSpec((NRB, 1), lambda i: (i, 0)),
            pl.BlockSpec((NRB, D), lambda i: (i, 0)),
        ],
        out_shape=[
            jax.ShapeDtypeStruct((N_ROWS, 1), jnp.float32),
            jax.ShapeDtypeStruct((N_ROWS, D), jnp.bfloat16),
        ],
    )(xg, role_indices.reshape(-1, 1).astype(jnp.int32), role_table, Wq.T,
      bq.reshape(-1, 1), Wk, _row(bk), Wv, _row(bv), gctx.reshape(-1, 1))

    w, topk = pl.pallas_call(
        _route_body,
        grid=(B // BB,),
        in_specs=[
            pl.BlockSpec((BB, L), lambda i: (i, 0)),
            pl.BlockSpec((BB, L, D), lambda i: (i, 0, 0)),
            full((1, D)), full((1, D)), full((D, E)), full((1, E)),
            full((1, E)),
        ],
        out_specs=[
            pl.BlockSpec((BB, E), lambda i: (i, 0)),
            pl.BlockSpec((BB, K), lambda i: (i, 0)),
        ],
        out_shape=[
            jax.ShapeDtypeStruct((B, E), jnp.float32),
            jax.ShapeDtypeStruct((B, K), jnp.int32),
        ],
    )(score_col.reshape(B, L), values.reshape(B, L, D), _row(ln_g),
      _row(ln_b), Wg, _row(bg), _row(expert_biases))

    logits = pl.pallas_call(
        _expert_body,
        grid=(E,),
        in_specs=[
            full((B, F)), full((B, E)),
            pl.BlockSpec((1, F, H1), lambda e: (e, 0, 0)),
            pl.BlockSpec((1, 1, H1), lambda e: (e, 0, 0)),
            pl.BlockSpec((1, H1, H2), lambda e: (e, 0, 0)),
            pl.BlockSpec((1, 1, H2), lambda e: (e, 0, 0)),
            pl.BlockSpec((1, H2, O), lambda e: (e, 0, 0)),
            pl.BlockSpec((1, 1, O), lambda e: (e, 0, 0)),
            full((O, C)), full((1, C)),
        ],
        out_specs=pl.BlockSpec((B, C), lambda e: (0, 0)),
        out_shape=jax.ShapeDtypeStruct((B, C), jnp.float32),
        scratch_shapes=[pltpu.VMEM((B, O), jnp.float32)],
    )(bibliometric_features, w, W1, b1.reshape(E, 1, H1), W2,
      b2.reshape(E, 1, H2), W3, b3.reshape(E, 1, O), Wp, _row(bp))

    return (logits, topk)


# speedup vs baseline: 1.0576x; 1.0576x over previous
import functools

import jax
import jax.numpy as jnp
from jax import lax
from jax.experimental import pallas as pl
from jax.experimental.pallas import tpu as pltpu
from jax.experimental.pallas import tpu_sc as plsc

B = 1024
L = 50
V = 100000
R = 16
D = 128
F = 256
E = 8
K = 2
H1 = 512
H2 = 512
O = 256
C = 16

N_ROWS = B * L
_NC = 2
_NS = 16
_NW = _NC * _NS
ROWS_PER_W = N_ROWS // _NW
CHUNK = 80
NCHUNK = ROWS_PER_W // CHUNK

_HI = lax.Precision.HIGHEST


def _b16(a):
    return a.astype(jnp.bfloat16).astype(jnp.float32)


NBUF = 4


def _sc_gather_body(table_hbm, idx_hbm, out_hbm, *scr):
    wid = lax.axis_index("s") * _NC + lax.axis_index("c")
    base = wid * ROWS_PER_W
    idxs = scr[0:NBUF]
    rows = scr[NBUF:2 * NBUF]
    gsems = scr[2 * NBUF:3 * NBUF]
    wsems = scr[3 * NBUF:4 * NBUF]

    def fire(j, s):
        off = base + j * CHUNK
        pltpu.sync_copy(idx_hbm.at[pl.ds(off, CHUNK)], idxs[s])
        return pltpu.async_copy(table_hbm.at[idxs[s]], rows[s], gsems[s])

    cps = [None] * NBUF
    wcs = [None] * NBUF
    for j in range(min(NBUF - 1, NCHUNK)):
        cps[j] = fire(j, j)
    for j in range(NCHUNK):
        s = j % NBUF
        cps[s].wait()
        wcs[s] = pltpu.async_copy(
            rows[s], out_hbm.at[pl.ds(base + j * CHUNK, CHUNK)], wsems[s])
        nj = j + NBUF - 1
        if nj < NCHUNK:
            sn = nj % NBUF
            if wcs[sn] is not None:
                wcs[sn].wait()
            cps[sn] = fire(nj, sn)
    for s in range(NBUF):
        if wcs[s] is not None:
            wcs[s].wait()


@functools.cache
def _build_sc_gather():
    return functools.partial(
        pl.kernel,
        mesh=plsc.VectorSubcoreMesh(core_axis_name="c",
                                    subcore_axis_name="s"),
        out_type=jax.ShapeDtypeStruct((N_ROWS, D), jnp.float32),
        scratch_types=(
            [pltpu.VMEM((CHUNK,), jnp.int32) for _ in range(NBUF)]
            + [pltpu.VMEM((CHUNK, D), jnp.float32) for _ in range(NBUF)]
            + [pltpu.SemaphoreType.DMA for _ in range(2 * NBUF)]
        ),
    )(_sc_gather_body)


NRB = 3200


def _proj_body(xg_ref, ridx_ref, rt_ref, Wq_ref, bq_ref, Wk_ref, bk_ref,
               Wv_ref, bv_ref, gctx_ref, score_ref, values_ref):
    f32 = jnp.float32
    oh = (ridx_ref[:] == lax.broadcasted_iota(jnp.int32, (1, R), 1))
    role = jnp.dot(oh.astype(f32), rt_ref[:], preferred_element_type=f32,
                   precision=_HI)
    x2 = xg_ref[:] + role
    q = jnp.dot(gctx_ref[:], Wq_ref[:], preferred_element_type=f32) \
        + bq_ref[:]
    keys = jnp.dot(x2, Wk_ref[:], preferred_element_type=f32) + bk_ref[:]
    score_ref[:] = jnp.sum(_b16(keys) * _b16(q), axis=1, keepdims=True)
    values_ref[:] = (jnp.dot(x2, Wv_ref[:], preferred_element_type=f32)
                     + bv_ref[:]).astype(jnp.bfloat16)


BB = 128


def _route_body(score_ref, val_ref, lng_ref, lnb_ref, Wg_ref, bg_ref,
                eb_ref, w_ref, topk_ref):
    f32 = jnp.float32
    score = score_ref[:]
    m = jnp.max(score, axis=1, keepdims=True)
    ex = jnp.exp(score - m)
    attn = ex / jnp.sum(ex, axis=1, keepdims=True)
    attn16 = _b16(attn)
    parts = [jnp.zeros((BB, D), f32) for _ in range(4)]
    for l in range(L):
        parts[l % 4] = (parts[l % 4]
                        + val_ref[:, l, :].astype(f32) * attn16[:, l:l + 1])
    pooled = (parts[0] + parts[1]) + (parts[2] + parts[3])
    mu = jnp.mean(pooled, axis=1, keepdims=True)
    cent = pooled - mu
    var = jnp.mean(cent * cent, axis=1, keepdims=True)
    ri = cent / jnp.sqrt(var + 1e-5) * lng_ref[:] + lnb_ref[:]

    gate = jnp.dot(ri, Wg_ref[:], preferred_element_type=f32) + bg_ref[:]
    sel = gate + eb_ref[:]
    iota_e = lax.broadcasted_iota(jnp.int32, (BB, E), 1)
    m1 = jnp.max(sel, axis=1, keepdims=True)
    i1 = jnp.min(jnp.where(sel == m1, iota_e, E), axis=1, keepdims=True)
    sel2 = jnp.where(iota_e == i1, -jnp.inf, sel)
    m2 = jnp.max(sel2, axis=1, keepdims=True)
    i2 = jnp.min(jnp.where(sel2 == m2, iota_e, E), axis=1, keepdims=True)
    g1 = jnp.sum(jnp.where(iota_e == i1, gate, 0.0), axis=1, keepdims=True)
    g2 = jnp.sum(jnp.where(iota_e == i2, gate, 0.0), axis=1, keepdims=True)
    mx = jnp.maximum(g1, g2)
    e1 = jnp.exp(g1 - mx)
    e2 = jnp.exp(g2 - mx)
    p1 = e1 / (e1 + e2)
    p2 = e2 / (e1 + e2)
    w_ref[:] = (p1 * (iota_e == i1).astype(f32)
                + p2 * (iota_e == i2).astype(f32))
    topk_ref[:] = jnp.concatenate([i1, i2], axis=1)


def _expert_body(feat_ref, w_ref, W1_ref, b1_ref, W2_ref, b2_ref,
                 W3_ref, b3_ref, Wp_ref, bp_ref, logits_ref, acc_ref):
    f32 = jnp.float32
    e = pl.program_id(0)
    h1 = jnp.maximum(
        jnp.dot(feat_ref[:], W1_ref[0], preferred_element_type=f32)
        + b1_ref[0], 0.0)
    h2 = jnp.maximum(
        jnp.dot(h1, W2_ref[0], preferred_element_type=f32) + b2_ref[0], 0.0)
    h3 = jnp.dot(h2, W3_ref[0], preferred_element_type=f32) + b3_ref[0]
    iota_e = lax.broadcasted_iota(jnp.int32, (B, E), 1)
    wcol = jnp.sum(jnp.where(iota_e == e, w_ref[:], 0.0), axis=1,
                   keepdims=True)

    @pl.when(e == 0)
    def _():
        acc_ref[:] = jnp.zeros((B, O), f32)

    acc_ref[:] += wcol * h3

    @pl.when(e == E - 1)
    def _():
        logits_ref[:] = (jnp.dot(acc_ref[:], Wp_ref[:],
                                 preferred_element_type=f32) + bp_ref[:])


def _row(v):
    return v.reshape(1, -1)


def kernel(ipc_indices, role_indices, bibliometric_features, ipc_table,
           role_table, Wq, bq, Wk, bk, Wv, bv, gctx, ln_g, ln_b, Wg, bg,
           expert_biases, W1, b1, W2, b2, W3, b3, Wp, bp):
    idx_flat = ipc_indices.reshape(-1).astype(jnp.int32)
    xg = _build_sc_gather()(ipc_table, idx_flat)

    full = lambda shape: pl.BlockSpec(shape, lambda i: (0,) * len(shape))
    score_col, values = pl.pallas_call(
        _proj_body,
        grid=(N_ROWS // NRB,),
        in_specs=[
            pl.BlockSpec((NRB, D), lambda i: (i, 0)),
            pl.BlockSpec((NRB, 1), lambda i: (i, 0)),
            full((R, D)), full((D, D)), full((1, D)), full((D, D)),
            full((1, D)), full((D, D)), full((1, D)), full((1, D)),
        ],
        out_specs=[
            pl.BlockSpec((NRB, 1), lambda i: (i, 0)),
            pl.BlockSpec((NRB, D), lambda i: (i, 0)),
        ],
        out_shape=[
            jax.ShapeDtypeStruct((N_ROWS, 1), jnp.float32),
            jax.ShapeDtypeStruct((N_ROWS, D), jnp.bfloat16),
        ],
    )(xg, role_indices.reshape(-1, 1).astype(jnp.int32), role_table, Wq,
      _row(bq), Wk, _row(bk), Wv, _row(bv), _row(gctx))

    w, topk = pl.pallas_call(
        _route_body,
        grid=(B // BB,),
        in_specs=[
            pl.BlockSpec((BB, L), lambda i: (i, 0)),
            pl.BlockSpec((BB, L, D), lambda i: (i, 0, 0)),
            full((1, D)), full((1, D)), full((D, E)), full((1, E)),
            full((1, E)),
        ],
        out_specs=[
            pl.BlockSpec((BB, E), lambda i: (i, 0)),
            pl.BlockSpec((BB, K), lambda i: (i, 0)),
        ],
        out_shape=[
            jax.ShapeDtypeStruct((B, E), jnp.float32),
            jax.ShapeDtypeStruct((B, K), jnp.int32),
        ],
    )(score_col.reshape(B, L), values.reshape(B, L, D), _row(ln_g),
      _row(ln_b), Wg, _row(bg), _row(expert_biases))

    logits = pl.pallas_call(
        _expert_body,
        grid=(E,),
        in_specs=[
            full((B, F)), full((B, E)),
            pl.BlockSpec((1, F, H1), lambda e: (e, 0, 0)),
            pl.BlockSpec((1, 1, H1), lambda e: (e, 0, 0)),
            pl.BlockSpec((1, H1, H2), lambda e: (e, 0, 0)),
            pl.BlockSpec((1, 1, H2), lambda e: (e, 0, 0)),
            pl.BlockSpec((1, H2, O), lambda e: (e, 0, 0)),
            pl.BlockSpec((1, 1, O), lambda e: (e, 0, 0)),
            full((O, C)), full((1, C)),
        ],
        out_specs=pl.BlockSpec((B, C), lambda e: (0, 0)),
        out_shape=jax.ShapeDtypeStruct((B, C), jnp.float32),
        scratch_shapes=[pltpu.VMEM((B, O), jnp.float32)],
    )(bibliometric_features, w, W1, b1.reshape(E, 1, H1), W2,
      b2.reshape(E, 1, H2), W3, b3.reshape(E, 1, O), Wp, _row(bp))

    return (logits, topk)
